# Initial kernel scaffold; baseline (speedup 1.0000x reference)
import jax
import jax.numpy as jnp
from jax.experimental import pallas as pl


def kernel(x, enc_w0, enc_b0, enc_w1, enc_b1, enc_w2, enc_b2, enc_w3, enc_b3, enc_w4, enc_b4, dec_w0, dec_b0, dec_w1, dec_b1, dec_w2, dec_b2, dec_w3, dec_b3, dec_w4, dec_b4, dec_w5, dec_b5):
    raise NotImplementedError("write your pallas kernel here")


# speedup vs baseline: 1.0000x; 1.0000x over previous
import reference as _r

def kernel(*args):
    return _r.reference(*args)


# speedup vs baseline: 2.1347x; 2.1346x over previous
import functools

import numpy as np
import jax
import jax.numpy as jnp
from jax import lax
from jax.experimental import pallas as pl
from jax.experimental.pallas import tpu as pltpu

_EPS = 1e-5
_SLOPE = 0.2
_VMEM = 40 * 1024 * 1024


def _ceil_to(v, m):
    return (v + m - 1) // m * m


def _row_tile(rows):
    for t in (512, 256, 128, 64, 32, 16, 8):
        if rows % t == 0:
            return t
    return rows


def _apply_act(y, act):
    if act == 'relu':
        return jnp.maximum(y, 0.0)
    if act == 'leaky_relu':
        return jnp.where(y >= 0.0, y, _SLOPE * y)
    if act == 'sigmoid':
        return jax.nn.sigmoid(y)
    return y


def _mm_kernel(x_ref, w_ref, b_ref, o_ref, *, act):
    acc = jnp.dot(x_ref[...], w_ref[...], preferred_element_type=jnp.float32)
    o_ref[...] = _apply_act(acc + b_ref[...], act)


def _mm_stats_kernel(x_ref, w_ref, o_ref, st_ref):
    acc = jnp.dot(x_ref[...], w_ref[...], preferred_element_type=jnp.float32)
    o_ref[...] = acc
    st_ref[...] = jnp.concatenate(
        [jnp.sum(acc, axis=0, keepdims=True),
         jnp.sum(acc * acc, axis=0, keepdims=True),
         jnp.zeros((6, acc.shape[1]), jnp.float32)], axis=0)


def _affine_kernel(x_ref, sc_ref, of_ref, o_ref, *, act):
    o_ref[...] = _apply_act(x_ref[...] * sc_ref[...] + of_ref[...], act)


def _stats_kernel(x_ref, st_ref):
    x = x_ref[...]
    st_ref[...] = jnp.concatenate(
        [jnp.sum(x, axis=0, keepdims=True),
         jnp.sum(x * x, axis=0, keepdims=True),
         jnp.zeros((6, x.shape[1]), jnp.float32)], axis=0)


_CPARAMS = pltpu.CompilerParams(dimension_semantics=("parallel",),
                                vmem_limit_bytes=_VMEM)


def _matmul(x, w, bias=None, act=None, emit_stats=False):
    M, K = x.shape
    _, Nc = w.shape
    Kp = _ceil_to(K, 128)
    Np = _ceil_to(Nc, 128)
    tm = 1024 if Np <= 128 else (512 if Np <= 512 else 256)
    tm = min(tm, _ceil_to(M, 8))
    Mp = _ceil_to(M, tm)
    nt = Mp // tm

    xp = jnp.pad(x.astype(jnp.bfloat16), ((0, Mp - M), (0, Kp - K)))
    wp = jnp.pad(w.astype(jnp.bfloat16), ((0, Kp - K), (0, Np - Nc)))

    x_spec = pl.BlockSpec((tm, Kp), lambda i: (i, 0))
    w_spec = pl.BlockSpec((Kp, Np), lambda i: (0, 0))
    o_spec = pl.BlockSpec((tm, Np), lambda i: (i, 0))

    if emit_stats:
        out, st = pl.pallas_call(
            _mm_stats_kernel,
            out_shape=(jax.ShapeDtypeStruct((Mp, Np), jnp.float32),
                       jax.ShapeDtypeStruct((nt * 8, Np), jnp.float32)),
            grid=(nt,),
            in_specs=[x_spec, w_spec],
            out_specs=[o_spec, pl.BlockSpec((8, Np), lambda i: (i, 0))],
            compiler_params=_CPARAMS,
        )(xp, wp)
        st = st.reshape(nt, 8, Np)
        return out, st[:, 0, :].sum(axis=0), st[:, 1, :].sum(axis=0)

    b = jnp.zeros((Nc,), jnp.float32) if bias is None else bias.astype(jnp.float32)
    bp = jnp.pad(b.reshape(1, -1), ((0, 0), (0, Np - Nc)))
    out = pl.pallas_call(
        functools.partial(_mm_kernel, act=act),
        out_shape=jax.ShapeDtypeStruct((Mp, Np), jnp.float32),
        grid=(nt,),
        in_specs=[x_spec, w_spec, pl.BlockSpec((1, Np), lambda i: (0, 0))],
        out_specs=o_spec,
        compiler_params=_CPARAMS,
    )(xp, wp, bp)
    return out


def _affine(x2d, scale, offset, act):
    R, C = x2d.shape
    t = _row_tile(R)
    return pl.pallas_call(
        functools.partial(_affine_kernel, act=act),
        out_shape=jax.ShapeDtypeStruct((R, C), jnp.float32),
        grid=(R // t,),
        in_specs=[pl.BlockSpec((t, C), lambda i: (i, 0)),
                  pl.BlockSpec((1, C), lambda i: (0, 0)),
                  pl.BlockSpec((1, C), lambda i: (0, 0))],
        out_specs=pl.BlockSpec((t, C), lambda i: (i, 0)),
        compiler_params=_CPARAMS,
    )(x2d, scale.reshape(1, -1).astype(jnp.float32),
      offset.reshape(1, -1).astype(jnp.float32))


def _channel_stats(x2d):
    R, C = x2d.shape
    t = _row_tile(R)
    nt = R // t
    st = pl.pallas_call(
        _stats_kernel,
        out_shape=jax.ShapeDtypeStruct((nt * 8, C), jnp.float32),
        grid=(nt,),
        in_specs=[pl.BlockSpec((t, C), lambda i: (i, 0))],
        out_specs=pl.BlockSpec((8, C), lambda i: (i, 0)),
        compiler_params=_CPARAMS,
    )(x2d)
    st = st.reshape(nt, 8, C)
    return st[:, 0, :].sum(axis=0), st[:, 1, :].sum(axis=0)


def _bn_scale_offset(ssum, ssq, count):
    mean = ssum / count
    var = jnp.maximum(ssq / count - mean * mean, 0.0)
    inv = lax.rsqrt(var + _EPS)
    return inv, -mean * inv


_ENC = [
    dict(cin=1,   cout=64,  k=3, stride=2, bn=False, act='relu'),
    dict(cin=64,  cout=128, k=3, stride=2, bn=True,  act='leaky_relu'),
    dict(cin=128, cout=256, k=5, stride=3, bn=True,  act='leaky_relu'),
    dict(cin=256, cout=512, k=5, stride=2, bn=True,  act='leaky_relu'),
    dict(cin=512, cout=300, k=3, stride=2, bn=False, act='sigmoid'),
]
_DEC = [
    dict(cin=300, cout=512, k=3, stride=2, bn=True,  act='relu'),
    dict(cin=512, cout=256, k=5, stride=2, bn=True,  act='relu'),
    dict(cin=256, cout=128, k=5, stride=3, bn=True,  act='relu'),
    dict(cin=128, cout=64,  k=5, stride=2, bn=True,  act='relu'),
    dict(cin=64,  cout=16,  k=5, stride=2, bn=True,  act='relu'),
    dict(cin=16,  cout=1,   k=4, stride=1, bn=False, act='sigmoid'),
]


def _conv(x, w, b, cfg):
    k, s, cout, cin = cfg['k'], cfg['stride'], cfg['cout'], cfg['cin']
    N, H, W, _ = x.shape
    Ho = (H - k) // s + 1
    Wo = (W - k) // s + 1
    xb = x.astype(jnp.bfloat16)
    cols = []
    for di in range(k):
        for dj in range(k):
            cols.append(lax.slice(
                xb, (0, di, dj, 0),
                (N, di + (Ho - 1) * s + 1, dj + (Wo - 1) * s + 1, cin),
                (1, s, s, 1)))
    patches = jnp.concatenate(cols, axis=-1).reshape(N * Ho * Wo, k * k * cin)
    M = patches.shape[0]
    w2 = jnp.transpose(w, (2, 3, 1, 0)).reshape(k * k * cin, cout)

    if cfg['bn']:
        out, ssum, ssq = _matmul(patches, w2, emit_stats=True)
        scale, offset = _bn_scale_offset(ssum, ssq, float(M))
        y = _affine(out, scale, offset, cfg['act'])
    else:
        y = _matmul(patches, w2, bias=b, act=cfg['act'])
    return y[:M, :cout].reshape(N, Ho, Wo, cout)


def _lane_dense(y, cout):
    M2 = int(np.prod(y.shape[:-1]))
    factor = 128 // cout if (cout < 128 and 128 % cout == 0) else 1
    lane_c = cout * factor
    rows = -(-M2 // factor)
    t = min(512, _ceil_to(rows, 8))
    rows_p = _ceil_to(rows, t)
    flat = jnp.pad(y.reshape(-1), (0, rows_p * lane_c - M2 * cout))
    return flat.reshape(rows_p, lane_c), factor, M2


def _conv_transpose(x, w, b, cfg):
    k, s, cin, cout = cfg['k'], cfg['stride'], cfg['cin'], cfg['cout']
    N, Hin, Win, _ = x.shape
    Hout = (Hin - 1) * s + k
    Wout = (Win - 1) * s + k
    Qy = Hin - 1 + -(-k // s)
    Qx = Win - 1 + -(-k // s)

    x2 = x.astype(jnp.bfloat16).reshape(N * Hin * Win, cin)
    w2 = jnp.transpose(w, (0, 2, 3, 1)).reshape(cin, k * k * cout)
    out = _matmul(x2, w2)
    contrib = out[:N * Hin * Win, :k * k * cout].reshape(N, Hin, Win, k, k, cout)

    phases = []
    for ry in range(s):
        for rx in range(s):
            acc = None
            for jy in range(-(-(k - ry) // s)):
                dy = ry + s * jy
                for jx in range(-(-(k - rx) // s)):
                    dx = rx + s * jx
                    term = jnp.pad(
                        contrib[:, :, :, dy, dx, :],
                        ((0, 0), (jy, Qy - Hin - jy), (jx, Qx - Win - jx),
                         (0, 0)))
                    acc = term if acc is None else acc + term
            phases.append(acc)
    y = jnp.stack(phases, axis=3).reshape(N, Qy, Qx, s, s, cout)
    y = y.transpose(0, 1, 3, 2, 4, 5).reshape(N, Qy * s, Qx * s, cout)
    y = y[:, :Hout, :Wout, :]

    flat, factor, M2 = _lane_dense(y, cout)
    if cfg['bn']:
        ssum_l, ssq_l = _channel_stats(flat)
        ssum = ssum_l.reshape(factor, cout).sum(axis=0)
        ssq = ssq_l.reshape(factor, cout).sum(axis=0)
        scale, offset = _bn_scale_offset(ssum, ssq, float(M2))
        scale_l = jnp.tile(scale, factor)
        offset_l = jnp.tile(offset, factor)
    else:
        bb = jnp.zeros((cout,), jnp.float32) if b is None else b.astype(jnp.float32)
        scale_l = jnp.ones((cout * factor,), jnp.float32)
        offset_l = jnp.tile(bb, factor)
    outf = _affine(flat, scale_l, offset_l, cfg['act'])
    return outf.reshape(-1)[:M2 * cout].reshape(N, Hout, Wout, cout)


def kernel(x,
           enc_w0, enc_b0, enc_w1, enc_b1, enc_w2, enc_b2,
           enc_w3, enc_b3, enc_w4, enc_b4,
           dec_w0, dec_b0, dec_w1, dec_b1, dec_w2, dec_b2,
           dec_w3, dec_b3, dec_w4, dec_b4, dec_w5, dec_b5):
    enc_p = [(enc_w0, enc_b0), (enc_w1, enc_b1), (enc_w2, enc_b2),
             (enc_w3, enc_b3), (enc_w4, enc_b4)]
    dec_p = [(dec_w0, dec_b0), (dec_w1, dec_b1), (dec_w2, dec_b2),
             (dec_w3, dec_b3), (dec_w4, dec_b4), (dec_w5, dec_b5)]
    h = jnp.transpose(x, (0, 2, 3, 1)).astype(jnp.float32)
    for (w, b), cfg in zip(enc_p, _ENC):
        h = _conv(h, w, b, cfg)
    encoding = h
    y = encoding
    for (w, b), cfg in zip(dec_p, _DEC):
        y = _conv_transpose(y, w, b, cfg)
    reconstruction = jnp.transpose(y, (0, 3, 1, 2))
    encoding_nchw = jnp.transpose(encoding, (0, 3, 1, 2))
    return reconstruction, encoding_nchw
